# baseline (speedup 1.0000x reference)
import functools

import jax
import jax.numpy as jnp
from jax import lax
from jax.experimental import pallas as pl
from jax.experimental.pallas import tpu as pltpu
from jax.experimental.pallas import tpu_sc as plsc

_D = 1024
_L = 16
_NC = 2
_NS = 16
_NW = _NC * _NS
_ROWS = 4 * 8192

_SC_ROWS = 12288
_TC_ROWS = _ROWS - _SC_ROWS
_SC_BASE = _TC_ROWS

_RPW = _SC_ROWS // _NW
_CHUNK = 16
_NCHUNK = _RPW // _CHUNK

_TC_BLOCK = 1024


_GATHER_DNUMS = lax.GatherDimensionNumbers(
    offset_dims=(), collapsed_slice_dims=(0,), start_index_map=(0,))


def _shuffle(v, idx):
    return lax.gather(v, idx[:, None], _GATHER_DNUMS, slice_sizes=(1,),
                      mode=lax.GatherScatterMode.PROMISE_IN_BOUNDS)


def _allsum(v):
    for sh in (1, 2, 4, 8):
        idx = lax.iota(jnp.int32, _L) ^ sh
        v = v + _shuffle(v, idx)
    return v


def _row_block(numv, cntv, xrb, xb):
    def row(r, carry):
        numv, cntv = carry
        rs = xb[r, pl.ds(0, _L)]
        dv = jnp.abs(xrb[r, pl.ds(0, _L)] - rs)
        for j in range(1, _D // _L):
            xv = xb[r, pl.ds(j * _L, _L)]
            rv = xrb[r, pl.ds(j * _L, _L)]
            rs = rs + xv
            dv = dv + jnp.abs(rv - xv)
        tot = _allsum(rs)
        w = jnp.sign(tot)
        w = w * w
        numv = numv + dv * w
        cntv = cntv + w
        return numv, cntv

    return lax.fori_loop(0, _CHUNK, row, (numv, cntv))


_mesh = plsc.VectorSubcoreMesh(core_axis_name="c", subcore_axis_name="s")


@functools.partial(
    pl.kernel,
    mesh=_mesh,
    out_type=[
        jax.ShapeDtypeStruct((_NW, _L), jnp.float32),
        jax.ShapeDtypeStruct((_NW, _L), jnp.float32),
    ],
    scratch_types=[
        pltpu.VMEM((_CHUNK, _D), jnp.float32),
        pltpu.VMEM((_CHUNK, _D), jnp.float32),
        pltpu.VMEM((_CHUNK, _D), jnp.float32),
        pltpu.VMEM((_CHUNK, _D), jnp.float32),
        pltpu.VMEM((_L,), jnp.float32),
        pltpu.VMEM((_L,), jnp.float32),
        pltpu.SemaphoreType.DMA,
        pltpu.SemaphoreType.DMA,
        pltpu.SemaphoreType.DMA,
        pltpu.SemaphoreType.DMA,
    ],
)
def _sc_loss(xr_hbm, x_hbm, num_hbm, cnt_hbm,
             xr0, x0, xr1, x1, numst, cntst, s0, s1, s2, s3):
    wid = lax.axis_index("s") * _NC + lax.axis_index("c")
    base = _SC_BASE + wid * _RPW
    bufs = ((xr0, x0, s0, s1), (xr1, x1, s2, s3))

    def start(c, b):
        xrb, xb, sa, sb = bufs[b]
        r0 = base + c * _CHUNK
        pltpu.make_async_copy(xr_hbm.at[pl.ds(r0, _CHUNK), :], xrb, sa).start()
        pltpu.make_async_copy(x_hbm.at[pl.ds(r0, _CHUNK), :], xb, sb).start()

    def wait(b):
        xrb, xb, sa, sb = bufs[b]
        pltpu.make_async_copy(xr_hbm.at[pl.ds(0, _CHUNK), :], xrb, sa).wait()
        pltpu.make_async_copy(x_hbm.at[pl.ds(0, _CHUNK), :], xb, sb).wait()

    start(0, 0)

    def body(i, carry):
        numv, cntv = carry
        start(2 * i + 1, 1)
        wait(0)
        numv, cntv = _row_block(numv, cntv, xr0, x0)

        @pl.when(i < _NCHUNK // 2 - 1)
        def _():
            start(2 * i + 2, 0)

        wait(1)
        numv, cntv = _row_block(numv, cntv, xr1, x1)
        return numv, cntv

    numv = jnp.zeros((_L,), jnp.float32)
    cntv = jnp.zeros((_L,), jnp.float32)
    numv, cntv = lax.fori_loop(0, _NCHUNK // 2, body, (numv, cntv))

    numst[...] = numv
    cntst[...] = cntv
    pltpu.sync_copy(numst, num_hbm.at[wid])
    pltpu.sync_copy(cntst, cnt_hbm.at[wid])


def _tc_loss_kernel(xr_ref, x_ref, num_ref, cnt_ref):
    step = pl.program_id(0)

    @pl.when(step == 0)
    def _init():
        num_ref[0, 0] = 0.0
        cnt_ref[0, 0] = 0.0

    x = x_ref[...]
    xr = xr_ref[...]
    row_sum = jnp.sum(x, axis=1)
    mask = (row_sum != 0).astype(jnp.float32)
    absdiff_row = jnp.sum(jnp.abs(xr - x), axis=1)
    num_ref[0, 0] += jnp.sum(absdiff_row * mask)
    cnt_ref[0, 0] += jnp.sum(mask)


def _tc_loss(xr2, x2):
    return pl.pallas_call(
        _tc_loss_kernel,
        grid=(_TC_ROWS // _TC_BLOCK,),
        in_specs=[
            pl.BlockSpec((_TC_BLOCK, _D), lambda i: (i, 0)),
            pl.BlockSpec((_TC_BLOCK, _D), lambda i: (i, 0)),
        ],
        out_specs=[
            pl.BlockSpec((1, 1), lambda i: (0, 0), memory_space=pltpu.SMEM),
            pl.BlockSpec((1, 1), lambda i: (0, 0), memory_space=pltpu.SMEM),
        ],
        out_shape=[
            jax.ShapeDtypeStruct((1, 1), jnp.float32),
            jax.ShapeDtypeStruct((1, 1), jnp.float32),
        ],
        compiler_params=pltpu.CompilerParams(
            dimension_semantics=("arbitrary",),
        ),
    )(xr2, x2)


def kernel(x_rec, x):
    margin = 0.5
    B, L, D = x.shape
    xr2 = x_rec.reshape(B * L, D)
    x2 = x.reshape(B * L, D)
    num_sc, cnt_sc = _sc_loss(xr2, x2)
    num_tc, cnt_tc = _tc_loss(xr2, x2)
    num = num_tc[0, 0] + jnp.sum(num_sc)
    cnt = cnt_tc[0, 0] + jnp.sum(cnt_sc) / _L
    return num / (cnt * D) + margin


# speedup vs baseline: 1.0258x; 1.0258x over previous
import functools

import jax
import jax.numpy as jnp
from jax import lax
from jax.experimental import pallas as pl
from jax.experimental.pallas import tpu as pltpu
from jax.experimental.pallas import tpu_sc as plsc

_D = 1024
_L = 16
_NC = 2
_NS = 16
_NW = _NC * _NS
_ROWS = 4 * 8192

_SC_ROWS = 4096
_TC_ROWS = _ROWS - _SC_ROWS
_SC_BASE = _TC_ROWS

_RPW = _SC_ROWS // _NW
_CHUNK = 16
_NCHUNK = _RPW // _CHUNK

_TC_BLOCK = 1024
_JU = 8


_GATHER_DNUMS = lax.GatherDimensionNumbers(
    offset_dims=(), collapsed_slice_dims=(0,), start_index_map=(0,))


def _shuffle(v, idx):
    return lax.gather(v, idx[:, None], _GATHER_DNUMS, slice_sizes=(1,),
                      mode=lax.GatherScatterMode.PROMISE_IN_BOUNDS)


def _allsum(v):
    for sh in (1, 2, 4, 8):
        idx = lax.iota(jnp.int32, _L) ^ sh
        v = v + _shuffle(v, idx)
    return v


def _row_block(numv, cntv, xrb, xb):
    def row(r, carry):
        numv, cntv = carry

        def jblk(jo, c2):
            rs, dv = c2
            for ji in range(_JU):
                off = jo * (_JU * _L) + ji * _L
                xv = xb[r, pl.ds(off, _L)]
                rv = xrb[r, pl.ds(off, _L)]
                rs = rs + xv
                dv = dv + jnp.abs(rv - xv)
            return rs, dv

        zero = jnp.zeros((_L,), jnp.float32)
        rs, dv = lax.fori_loop(0, _D // (_JU * _L), jblk, (zero, zero))
        tot = _allsum(rs)
        w = jnp.sign(tot)
        w = w * w
        numv = numv + dv * w
        cntv = cntv + w
        return numv, cntv

    return lax.fori_loop(0, _CHUNK, row, (numv, cntv))


_mesh = plsc.VectorSubcoreMesh(core_axis_name="c", subcore_axis_name="s")


@functools.partial(
    pl.kernel,
    mesh=_mesh,
    out_type=[
        jax.ShapeDtypeStruct((_NW, _L), jnp.float32),
        jax.ShapeDtypeStruct((_NW, _L), jnp.float32),
    ],
    scratch_types=[
        pltpu.VMEM((_CHUNK, _D), jnp.float32),
        pltpu.VMEM((_CHUNK, _D), jnp.float32),
        pltpu.VMEM((_CHUNK, _D), jnp.float32),
        pltpu.VMEM((_CHUNK, _D), jnp.float32),
        pltpu.VMEM((_L,), jnp.float32),
        pltpu.VMEM((_L,), jnp.float32),
        pltpu.SemaphoreType.DMA,
        pltpu.SemaphoreType.DMA,
        pltpu.SemaphoreType.DMA,
        pltpu.SemaphoreType.DMA,
    ],
)
def _sc_loss(xr_hbm, x_hbm, num_hbm, cnt_hbm,
             xr0, x0, xr1, x1, numst, cntst, s0, s1, s2, s3):
    wid = lax.axis_index("s") * _NC + lax.axis_index("c")
    base = _SC_BASE + wid * _RPW
    bufs = ((xr0, x0, s0, s1), (xr1, x1, s2, s3))

    def start(c, b):
        xrb, xb, sa, sb = bufs[b]
        r0 = base + c * _CHUNK
        pltpu.make_async_copy(xr_hbm.at[pl.ds(r0, _CHUNK), :], xrb, sa).start()
        pltpu.make_async_copy(x_hbm.at[pl.ds(r0, _CHUNK), :], xb, sb).start()

    def wait(b):
        xrb, xb, sa, sb = bufs[b]
        pltpu.make_async_copy(xr_hbm.at[pl.ds(0, _CHUNK), :], xrb, sa).wait()
        pltpu.make_async_copy(x_hbm.at[pl.ds(0, _CHUNK), :], xb, sb).wait()

    start(0, 0)

    def body(i, carry):
        numv, cntv = carry
        start(2 * i + 1, 1)
        wait(0)
        numv, cntv = _row_block(numv, cntv, xr0, x0)

        @pl.when(i < _NCHUNK // 2 - 1)
        def _():
            start(2 * i + 2, 0)

        wait(1)
        numv, cntv = _row_block(numv, cntv, xr1, x1)
        return numv, cntv

    numv = jnp.zeros((_L,), jnp.float32)
    cntv = jnp.zeros((_L,), jnp.float32)
    numv, cntv = lax.fori_loop(0, _NCHUNK // 2, body, (numv, cntv))

    numst[...] = numv
    cntst[...] = cntv
    pltpu.sync_copy(numst, num_hbm.at[wid])
    pltpu.sync_copy(cntst, cnt_hbm.at[wid])


def _tc_loss_kernel(xr_ref, x_ref, num_ref, cnt_ref):
    step = pl.program_id(0)

    @pl.when(step == 0)
    def _init():
        num_ref[0, 0] = 0.0
        cnt_ref[0, 0] = 0.0

    x = x_ref[...]
    xr = xr_ref[...]
    row_sum = jnp.sum(x, axis=1)
    mask = (row_sum != 0).astype(jnp.float32)
    absdiff_row = jnp.sum(jnp.abs(xr - x), axis=1)
    num_ref[0, 0] += jnp.sum(absdiff_row * mask)
    cnt_ref[0, 0] += jnp.sum(mask)


def _tc_loss(xr2, x2):
    return pl.pallas_call(
        _tc_loss_kernel,
        grid=(_TC_ROWS // _TC_BLOCK,),
        in_specs=[
            pl.BlockSpec((_TC_BLOCK, _D), lambda i: (i, 0)),
            pl.BlockSpec((_TC_BLOCK, _D), lambda i: (i, 0)),
        ],
        out_specs=[
            pl.BlockSpec((1, 1), lambda i: (0, 0), memory_space=pltpu.SMEM),
            pl.BlockSpec((1, 1), lambda i: (0, 0), memory_space=pltpu.SMEM),
        ],
        out_shape=[
            jax.ShapeDtypeStruct((1, 1), jnp.float32),
            jax.ShapeDtypeStruct((1, 1), jnp.float32),
        ],
        compiler_params=pltpu.CompilerParams(
            dimension_semantics=("arbitrary",),
        ),
    )(xr2, x2)


def kernel(x_rec, x):
    margin = 0.5
    B, L, D = x.shape
    xr2 = x_rec.reshape(B * L, D)
    x2 = x.reshape(B * L, D)
    num_sc, cnt_sc = _sc_loss(xr2, x2)
    num_tc, cnt_tc = _tc_loss(xr2, x2)
    num = num_tc[0, 0] + jnp.sum(num_sc)
    cnt = cnt_tc[0, 0] + jnp.sum(cnt_sc) / _L
    return num / (cnt * D) + margin


# speedup vs baseline: 1.0341x; 1.0081x over previous
import functools

import jax
import jax.numpy as jnp
from jax import lax
from jax.experimental import pallas as pl
from jax.experimental.pallas import tpu as pltpu
from jax.experimental.pallas import tpu_sc as plsc

_D = 1024
_L = 16
_NC = 2
_NS = 16
_NW = _NC * _NS
_ROWS = 4 * 8192

_SC_ROWS = 4096
_TC_ROWS = _ROWS - _SC_ROWS
_SC_BASE = _TC_ROWS

_RPW = _SC_ROWS // _NW
_CHUNK = 16
_NCHUNK = _RPW // _CHUNK

_TC_BLOCK = 1024
_JU = 8


_GATHER_DNUMS = lax.GatherDimensionNumbers(
    offset_dims=(), collapsed_slice_dims=(0,), start_index_map=(0,))


def _shuffle(v, idx):
    return lax.gather(v, idx[:, None], _GATHER_DNUMS, slice_sizes=(1,),
                      mode=lax.GatherScatterMode.PROMISE_IN_BOUNDS)


def _allsum(v):
    for sh in (1, 2, 4, 8):
        idx = lax.iota(jnp.int32, _L) ^ sh
        v = v + _shuffle(v, idx)
    return v


def _row_block(numv, cntv, xrb, xb):
    def row(r, carry):
        numv, cntv = carry

        def jblk(jo, c2):
            rs, dv = c2
            for ji in range(_JU):
                off = jo * (_JU * _L) + ji * _L
                xv = xb[r, pl.ds(off, _L)]
                rv = xrb[r, pl.ds(off, _L)]
                rs = rs + xv
                dv = dv + jnp.abs(rv - xv)
            return rs, dv

        zero = jnp.zeros((_L,), jnp.float32)
        rs, dv = lax.fori_loop(0, _D // (_JU * _L), jblk, (zero, zero))
        tot = _allsum(rs)
        w = jnp.sign(tot)
        w = w * w
        numv = numv + dv * w
        cntv = cntv + w
        return numv, cntv

    return lax.fori_loop(0, _CHUNK, row, (numv, cntv))


_mesh = plsc.VectorSubcoreMesh(core_axis_name="c", subcore_axis_name="s")


@functools.partial(
    pl.kernel,
    mesh=_mesh,
    out_type=jax.ShapeDtypeStruct((2 * _NW, _L), jnp.float32),
    scratch_types=[
        pltpu.VMEM((_CHUNK, _D), jnp.float32),
        pltpu.VMEM((_CHUNK, _D), jnp.float32),
        pltpu.VMEM((_CHUNK, _D), jnp.float32),
        pltpu.VMEM((_CHUNK, _D), jnp.float32),
        pltpu.VMEM((_L,), jnp.float32),
        pltpu.VMEM((_L,), jnp.float32),
        pltpu.SemaphoreType.DMA,
        pltpu.SemaphoreType.DMA,
        pltpu.SemaphoreType.DMA,
        pltpu.SemaphoreType.DMA,
    ],
)
def _sc_loss(xr_hbm, x_hbm, out_hbm,
             xr0, x0, xr1, x1, numst, cntst, s0, s1, s2, s3):
    wid = lax.axis_index("s") * _NC + lax.axis_index("c")
    base = _SC_BASE + wid * _RPW
    bufs = ((xr0, x0, s0, s1), (xr1, x1, s2, s3))

    def start(c, b):
        xrb, xb, sa, sb = bufs[b]
        r0 = base + c * _CHUNK
        pltpu.make_async_copy(xr_hbm.at[pl.ds(r0, _CHUNK), :], xrb, sa).start()
        pltpu.make_async_copy(x_hbm.at[pl.ds(r0, _CHUNK), :], xb, sb).start()

    def wait(b):
        xrb, xb, sa, sb = bufs[b]
        pltpu.make_async_copy(xr_hbm.at[pl.ds(0, _CHUNK), :], xrb, sa).wait()
        pltpu.make_async_copy(x_hbm.at[pl.ds(0, _CHUNK), :], xb, sb).wait()

    start(0, 0)

    def body(i, carry):
        numv, cntv = carry
        start(2 * i + 1, 1)
        wait(0)
        numv, cntv = _row_block(numv, cntv, xr0, x0)

        @pl.when(i < _NCHUNK // 2 - 1)
        def _():
            start(2 * i + 2, 0)

        wait(1)
        numv, cntv = _row_block(numv, cntv, xr1, x1)
        return numv, cntv

    numv = jnp.zeros((_L,), jnp.float32)
    cntv = jnp.zeros((_L,), jnp.float32)
    numv, cntv = lax.fori_loop(0, _NCHUNK // 2, body, (numv, cntv))

    numst[...] = numv
    cntst[...] = cntv
    pltpu.sync_copy(numst, out_hbm.at[wid])
    pltpu.sync_copy(cntst, out_hbm.at[_NW + wid])


def _tc_loss_kernel(xr_ref, x_ref, num_ref, cnt_ref):
    step = pl.program_id(0)

    @pl.when(step == 0)
    def _init():
        num_ref[0, 0] = 0.0
        cnt_ref[0, 0] = 0.0

    x = x_ref[...]
    xr = xr_ref[...]
    row_sum = jnp.sum(x, axis=1)
    mask = (row_sum != 0).astype(jnp.float32)
    absdiff_row = jnp.sum(jnp.abs(xr - x), axis=1)
    num_ref[0, 0] += jnp.sum(absdiff_row * mask)
    cnt_ref[0, 0] += jnp.sum(mask)


def _tc_loss(xr2, x2):
    return pl.pallas_call(
        _tc_loss_kernel,
        grid=(_TC_ROWS // _TC_BLOCK,),
        in_specs=[
            pl.BlockSpec((_TC_BLOCK, _D), lambda i: (i, 0)),
            pl.BlockSpec((_TC_BLOCK, _D), lambda i: (i, 0)),
        ],
        out_specs=[
            pl.BlockSpec((1, 1), lambda i: (0, 0), memory_space=pltpu.SMEM),
            pl.BlockSpec((1, 1), lambda i: (0, 0), memory_space=pltpu.SMEM),
        ],
        out_shape=[
            jax.ShapeDtypeStruct((1, 1), jnp.float32),
            jax.ShapeDtypeStruct((1, 1), jnp.float32),
        ],
        compiler_params=pltpu.CompilerParams(
            dimension_semantics=("arbitrary",),
        ),
    )(xr2, x2)


def kernel(x_rec, x):
    margin = 0.5
    B, L, D = x.shape
    xr2 = x_rec.reshape(B * L, D)
    x2 = x.reshape(B * L, D)
    sc_part = _sc_loss(xr2, x2)
    num_tc, cnt_tc = _tc_loss(xr2, x2)
    p = jnp.sum(sc_part.reshape(2, _NW * _L), axis=1)
    num = num_tc[0, 0] + p[0]
    cnt = cnt_tc[0, 0] + p[1] / _L
    return num / (cnt * D) + margin
